# baseline (speedup 1.0000x reference)
import functools

import jax
import jax.numpy as jnp
from jax import lax
from jax.experimental import pallas as pl
from jax.experimental.pallas import tpu as pltpu, tpu_sc as plsc

HIDDEN = 64
LANES = 16
NUM_CORES = 2
NUM_SUBCORES = 16
NW = NUM_CORES * NUM_SUBCORES

SEQ = 200
BATCH = 4096
ST = SEQ // 8
BT = BATCH // 128
SI_PER_UNIT = 4
UNIT_ROWS = SI_PER_UNIT * 128
N_UNITS = ST * BT * (8 // SI_PER_UNIT)
UPW = N_UNITS // NW
NBUF = 2


def _coords(ug):
    st = ug // 64
    rem = ug % 64
    j = rem // 2
    half = rem % 2
    return st, j, half


def _emb_body(seq_hbm, pid_hbm, tok_hbm, pos_hbm, out_hbm,
              idxb0, idxb1, pidxb0, pidxb1, trows0, trows1, hbuf,
              isem0, isem1, psem0, psem1, tsem0, tsem1, osem):
    idxb = [idxb0, idxb1]
    pidxb = [pidxb0, pidxb1]
    trows = [trows0, trows1]
    isem = [isem0, isem1]
    psem = [psem0, psem1]
    tsem = [tsem0, tsem1]

    wid = lax.axis_index("s") * NUM_CORES + lax.axis_index("c")
    u0 = wid * UPW

    iota16 = lax.iota(jnp.int32, LANES)
    bvec = [[iota16 + (sp * 128 + k * LANES) for k in range(8)]
            for sp in range(SI_PER_UNIT)]

    def stage(b, ug):
        st, j, half = _coords(ug)
        sl = pl.ds(SI_PER_UNIT * half, SI_PER_UNIT)
        pltpu.async_copy(seq_hbm.at[st, j, sl], idxb[b], isem[b])
        pltpu.async_copy(pid_hbm.at[st, j, sl], pidxb[b], isem[b])

    def wait_idx(b):
        pltpu.make_async_copy(seq_hbm.at[0, 0, pl.ds(0, SI_PER_UNIT)],
                              idxb[b], isem[b]).wait()
        pltpu.make_async_copy(pid_hbm.at[0, 0, pl.ds(0, SI_PER_UNIT)],
                              pidxb[b], isem[b]).wait()

    def start_pos(b):
        cps = [pltpu.async_copy(pos_hbm.at[pidxb[b].at[i]],
                                trows[b].at[pl.ds(128 * i, 128)], psem[b])
               for i in range(SI_PER_UNIT)]
        return cps

    def wait_out():
        pltpu.make_async_copy(
            hbuf, out_hbm.at[pl.ds(0, SI_PER_UNIT), slice(None), 0],
            osem).wait()

    stage(0, u0)
    wait_idx(0)
    start_pos(0)
    stage(1, u0 + 1)

    def group_body(g, carry):
        for b in range(NBUF):
            u = u0 + g * NBUF + b
            nxt = u + 1

            for i in range(SI_PER_UNIT):
                pltpu.make_async_copy(pos_hbm.at[pidxb[b].at[0]],
                                      trows[b].at[pl.ds(0, 128)],
                                      psem[b]).wait()
            tok_cps = [pltpu.async_copy(tok_hbm.at[idxb[b].at[i]],
                                        trows[b].at[pl.ds(128 * i, 128)],
                                        tsem[b], add=True)
                       for i in range(SI_PER_UNIT)]
            for cp in tok_cps:
                cp.wait()

            nb = (b + 1) % NBUF

            @pl.when(nxt < u0 + UPW)
            def _():
                wait_idx(nb)
                start_pos(nb)

            @pl.when(nxt + 1 < u0 + UPW)
            def _():
                stage(b, u + 2)

            @pl.when(u > u0)
            def _():
                wait_out()

            def ht_body(ht, c):
                for sp in range(SI_PER_UNIT):
                    for hi in range(8):
                        hvec = jnp.full((LANES,), ht * 8 + hi, jnp.int32)
                        for k in range(8):
                            vals = plsc.load_gather(trows[b], [bvec[sp][k], hvec])
                            hbuf[sp, ht, hi, pl.ds(k * LANES, LANES)] = vals
                return c

            lax.fori_loop(0, 8, ht_body, 0, unroll=False)

            st, j, half = _coords(u)
            pltpu.async_copy(
                hbuf,
                out_hbm.at[pl.ds(st * 8 + SI_PER_UNIT * half, SI_PER_UNIT),
                           slice(None), j],
                osem)
        return carry

    lax.fori_loop(0, UPW // NBUF, group_body, 0, unroll=False)
    wait_out()


def kernel(sequence, position_ids, token_table, pos_table):
    seq_r = sequence.T.reshape(ST, 8, BT, 128).transpose(0, 2, 1, 3).astype(jnp.int32)
    pid_r = position_ids.T.reshape(ST, 8, BT, 128).transpose(0, 2, 1, 3).astype(jnp.int32)

    mesh = plsc.VectorSubcoreMesh(core_axis_name="c", subcore_axis_name="s",
                                  num_cores=NUM_CORES, num_subcores=NUM_SUBCORES)
    scratch = (
        [pltpu.VMEM((SI_PER_UNIT, 128), jnp.int32) for _ in range(2 * NBUF)]
        + [pltpu.VMEM((UNIT_ROWS, HIDDEN), jnp.float32) for _ in range(NBUF)]
        + [pltpu.VMEM((SI_PER_UNIT, 8, 8, 128), jnp.float32)]
        + [pltpu.SemaphoreType.DMA for _ in range(3 * NBUF + 1)]
    )
    emb = functools.partial(
        pl.kernel,
        out_type=jax.ShapeDtypeStruct((SEQ, 8, BT, 8, 128), jnp.float32),
        mesh=mesh,
        scratch_types=scratch,
        compiler_params=pltpu.CompilerParams(use_tc_tiling_on_sc=False,
                                             needs_layout_passes=False),
    )(_emb_body)

    x = emb(seq_r, pid_r, token_table, pos_table)
    return x.transpose(2, 4, 0, 1, 3).reshape(BATCH, SEQ, HIDDEN)
